# baseline (speedup 1.0000x reference)
import functools

import jax
import jax.numpy as jnp
from jax import lax
from jax.experimental import pallas as pl
from jax.experimental.pallas import tpu as pltpu
from jax.experimental.pallas import tpu_sc as plsc

LATENT = 64
XDIM = 16
UDIM = LATENT + XDIM
LANES = 16
WIDTH = 128
IDX_CHUNK = 128
HALF = 256


def kernel(users, items, xij, emb_user, emb_item, emb_item_xij1, emb_item_xij0):
    B = users.shape[0]
    NI = emb_item.shape[0]
    info = plsc.get_sparse_core_info()
    NC, NS = info.num_cores, info.num_subcores
    NW = NC * NS
    assert B % (NW * LANES) == 0
    RPW = B // NW
    NCHUNK = RPW // IDX_CHUNK
    NHALF = RPW // HALF
    CPH = HALF // IDX_CHUNK

    us_p = jnp.pad(jax.nn.sigmoid(emb_user), ((0, 0), (0, WIDTH - UDIM)))
    icat = jnp.concatenate(
        [jnp.exp(emb_item), emb_item_xij1, emb_item_xij0,
         jnp.zeros((NI, WIDTH - LATENT - 2 * XDIM), jnp.float32)], axis=1)

    mesh = plsc.VectorSubcoreMesh(core_axis_name="c", subcore_axis_name="s")

    @functools.partial(
        pl.kernel,
        out_type=jax.ShapeDtypeStruct((B,), jnp.float32),
        mesh=mesh,
        scratch_types=[
            pltpu.VMEM((NCHUNK, IDX_CHUNK), jnp.int32),
            pltpu.VMEM((NCHUNK, IDX_CHUNK), jnp.int32),
            pltpu.VMEM((RPW,), jnp.float32),
            pltpu.VMEM((HALF, WIDTH), jnp.float32),
            pltpu.VMEM((HALF, WIDTH), jnp.float32),
            pltpu.VMEM((RPW,), jnp.float32),
            pltpu.SemaphoreType.DMA,
        ],
        compiler_params=pltpu.CompilerParams(
            needs_layout_passes=False, use_tc_tiling_on_sc=True),
    )
    def sc_kernel(users_h, items_h, xij_h, eu_h, ic_h, out_h,
                  uidx, iidx, xv, ubuf, ibuf, outv, sem):
        wid = lax.axis_index("s") * NC + lax.axis_index("c")
        base = wid * RPW

        for j in range(NCHUNK):
            sl = pl.ds(base + j * IDX_CHUNK, IDX_CHUNK)
            pltpu.sync_copy(users_h.at[sl], uidx.at[j])
            pltpu.sync_copy(items_h.at[sl], iidx.at[j])
        pltpu.sync_copy(xij_h.at[pl.ds(base, RPW)], xv)

        lanes = lax.broadcasted_iota(jnp.int32, (LANES,), 0)
        one = jnp.float32(1.0)

        for h in range(NHALF):
            copies = []
            for c in range(CPH):
                j = h * CPH + c
                sl = pl.ds(c * IDX_CHUNK, IDX_CHUNK)
                copies.append(pltpu.async_copy(eu_h.at[uidx.at[j]],
                                               ubuf.at[sl], sem))
                copies.append(pltpu.async_copy(ic_h.at[iidx.at[j]],
                                               ibuf.at[sl], sem))
            for cp in copies:
                cp.wait()

            def group_body(g, carry):
                rows = g * LANES + lanes
                x = xv[pl.ds(h * HALF + g * LANES, LANES)]
                denom = jnp.zeros((LANES,), jnp.float32)
                numer = jnp.zeros((LANES,), jnp.float32)
                for d in range(LATENT):
                    dd = jnp.full((LANES,), d, jnp.int32)
                    e = plsc.load_gather(ibuf, [rows, dd])
                    s = plsc.load_gather(ubuf, [rows, dd])
                    denom = denom + e
                    numer = numer + s * e
                for d in range(XDIM):
                    d1 = jnp.full((LANES,), LATENT + d, jnp.int32)
                    d0 = jnp.full((LANES,), LATENT + XDIM + d, jnp.int32)
                    x1 = plsc.load_gather(ibuf, [rows, d1])
                    x0 = plsc.load_gather(ibuf, [rows, d0])
                    e = jnp.exp(x1 * x + x0 * (one - x))
                    s = plsc.load_gather(ubuf, [rows, d1])
                    denom = denom + e
                    numer = numer + s * e
                outv[pl.ds(h * HALF + g * LANES, LANES)] = numer / denom
                return carry

            lax.fori_loop(0, HALF // LANES, group_body, 0)

        pltpu.sync_copy(outv, out_h.at[pl.ds(base, RPW)])

    return sc_kernel(users.astype(jnp.int32), items, xij, us_p, icat)


# speedup vs baseline: 5.6107x; 5.6107x over previous
import functools

import jax
import jax.numpy as jnp
from jax import lax
from jax.experimental import pallas as pl
from jax.experimental.pallas import tpu as pltpu
from jax.experimental.pallas import tpu_sc as plsc

LATENT = 64
XDIM = 16
UDIM = LATENT + XDIM
LANES = 16
WIDTH = 128
IDX_CHUNK = 128
HALF = 256


def kernel(users, items, xij, emb_user, emb_item, emb_item_xij1, emb_item_xij0):
    B = users.shape[0]
    NI = emb_item.shape[0]
    info = plsc.get_sparse_core_info()
    NC, NS = info.num_cores, info.num_subcores
    NW = NC * NS
    assert B % (NW * LANES) == 0
    RPW = B // NW
    NCHUNK = RPW // IDX_CHUNK
    NHALF = RPW // HALF
    CPH = HALF // IDX_CHUNK

    proj_u = jnp.eye(UDIM, WIDTH, dtype=jnp.float32)
    us_p = jax.nn.sigmoid(emb_user) @ proj_u
    proj_i = jnp.eye(3 * LATENT // 2, WIDTH, dtype=jnp.float32)
    icat = jnp.concatenate(
        [jnp.exp(emb_item), emb_item_xij1, emb_item_xij0], axis=1) @ proj_i

    mesh = plsc.VectorSubcoreMesh(core_axis_name="c", subcore_axis_name="s")

    @functools.partial(
        pl.kernel,
        out_type=jax.ShapeDtypeStruct((B,), jnp.float32),
        mesh=mesh,
        scratch_types=[
            pltpu.VMEM((NCHUNK, IDX_CHUNK), jnp.int32),
            pltpu.VMEM((NCHUNK, IDX_CHUNK), jnp.int32),
            pltpu.VMEM((RPW,), jnp.float32),
            pltpu.VMEM((HALF, WIDTH), jnp.float32),
            pltpu.VMEM((HALF, WIDTH), jnp.float32),
            pltpu.VMEM((RPW,), jnp.float32),
            pltpu.SemaphoreType.DMA,
        ],
        compiler_params=pltpu.CompilerParams(
            needs_layout_passes=False, use_tc_tiling_on_sc=True),
    )
    def sc_kernel(users_h, items_h, xij_h, eu_h, ic_h, out_h,
                  uidx, iidx, xv, ubuf, ibuf, outv, sem):
        wid = lax.axis_index("s") * NC + lax.axis_index("c")
        base = wid * RPW

        for j in range(NCHUNK):
            sl = pl.ds(base + j * IDX_CHUNK, IDX_CHUNK)
            pltpu.sync_copy(users_h.at[sl], uidx.at[j])
            pltpu.sync_copy(items_h.at[sl], iidx.at[j])
        pltpu.sync_copy(xij_h.at[pl.ds(base, RPW)], xv)

        lanes = lax.broadcasted_iota(jnp.int32, (LANES,), 0)
        one = jnp.float32(1.0)

        for h in range(NHALF):
            copies = []
            for c in range(CPH):
                j = h * CPH + c
                sl = pl.ds(c * IDX_CHUNK, IDX_CHUNK)
                copies.append(pltpu.async_copy(eu_h.at[uidx.at[j]],
                                               ubuf.at[sl], sem))
                copies.append(pltpu.async_copy(ic_h.at[iidx.at[j]],
                                               ibuf.at[sl], sem))
            for cp in copies:
                cp.wait()

            def group_body(g, carry):
                rows = g * LANES + lanes
                x = xv[pl.ds(h * HALF + g * LANES, LANES)]
                denom = jnp.zeros((LANES,), jnp.float32)
                numer = jnp.zeros((LANES,), jnp.float32)
                for d in range(LATENT):
                    dd = jnp.full((LANES,), d, jnp.int32)
                    e = plsc.load_gather(ibuf, [rows, dd])
                    s = plsc.load_gather(ubuf, [rows, dd])
                    denom = denom + e
                    numer = numer + s * e
                for d in range(XDIM):
                    d1 = jnp.full((LANES,), LATENT + d, jnp.int32)
                    d0 = jnp.full((LANES,), LATENT + XDIM + d, jnp.int32)
                    x1 = plsc.load_gather(ibuf, [rows, d1])
                    x0 = plsc.load_gather(ibuf, [rows, d0])
                    e = jnp.exp(x1 * x + x0 * (one - x))
                    s = plsc.load_gather(ubuf, [rows, d1])
                    denom = denom + e
                    numer = numer + s * e
                outv[pl.ds(h * HALF + g * LANES, LANES)] = numer / denom
                return carry

            lax.fori_loop(0, HALF // LANES, group_body, 0)

        pltpu.sync_copy(outv, out_h.at[pl.ds(base, RPW)])

    return sc_kernel(users.astype(jnp.int32), items, xij, us_p, icat)


# speedup vs baseline: 5.6542x; 1.0078x over previous
import functools

import jax
import jax.numpy as jnp
from jax import lax
from jax.experimental import pallas as pl
from jax.experimental.pallas import tpu as pltpu
from jax.experimental.pallas import tpu_sc as plsc

LATENT = 64
XDIM = 16
UDIM = LATENT + XDIM
LANES = 16
WIDTH = 128
IDX_CHUNK = 128
HALF = 256


def kernel(users, items, xij, emb_user, emb_item, emb_item_xij1, emb_item_xij0):
    B = users.shape[0]
    NI = emb_item.shape[0]
    info = plsc.get_sparse_core_info()
    NC, NS = info.num_cores, info.num_subcores
    NW = NC * NS
    assert B % (NW * LANES) == 0
    RPW = B // NW
    NCHUNK = RPW // IDX_CHUNK
    NHALF = RPW // HALF
    CPH = HALF // IDX_CHUNK

    proj_u = jnp.eye(UDIM, WIDTH, dtype=jnp.float32)
    us_p = jax.nn.sigmoid(emb_user) @ proj_u
    proj_i = jnp.eye(3 * LATENT // 2, WIDTH, dtype=jnp.float32)
    icat = jnp.concatenate(
        [jnp.exp(emb_item), emb_item_xij1, emb_item_xij0], axis=1) @ proj_i

    mesh = plsc.VectorSubcoreMesh(core_axis_name="c", subcore_axis_name="s")

    @functools.partial(
        pl.kernel,
        out_type=jax.ShapeDtypeStruct((B,), jnp.float32),
        mesh=mesh,
        scratch_types=[
            pltpu.VMEM((NCHUNK, IDX_CHUNK), jnp.int32),
            pltpu.VMEM((NCHUNK, IDX_CHUNK), jnp.int32),
            pltpu.VMEM((RPW,), jnp.float32),
            pltpu.VMEM((2 * IDX_CHUNK, WIDTH), jnp.float32),
            pltpu.VMEM((2 * IDX_CHUNK, WIDTH), jnp.float32),
            pltpu.VMEM((RPW,), jnp.float32),
            pltpu.SemaphoreType.DMA,
            pltpu.SemaphoreType.DMA,
        ],
        compiler_params=pltpu.CompilerParams(
            needs_layout_passes=False, use_tc_tiling_on_sc=True),
    )
    def sc_kernel(users_h, items_h, xij_h, eu_h, ic_h, out_h,
                  uidx, iidx, xv, ubuf, ibuf, outv, sem_a, sem_b):
        wid = lax.axis_index("s") * NC + lax.axis_index("c")
        base = wid * RPW

        for j in range(NCHUNK):
            sl = pl.ds(base + j * IDX_CHUNK, IDX_CHUNK)
            pltpu.sync_copy(users_h.at[sl], uidx.at[j])
            pltpu.sync_copy(items_h.at[sl], iidx.at[j])
        pltpu.sync_copy(xij_h.at[pl.ds(base, RPW)], xv)

        lanes = lax.broadcasted_iota(jnp.int32, (LANES,), 0)
        one = jnp.float32(1.0)

        def fire(q, sem):
            sl = pl.ds((q % 2) * IDX_CHUNK, IDX_CHUNK)
            cu = pltpu.async_copy(eu_h.at[uidx.at[q]], ubuf.at[sl], sem)
            ci = pltpu.async_copy(ic_h.at[iidx.at[q]], ibuf.at[sl], sem)
            return (cu, ci)

        def compute(q):
            def group_body(g, carry):
                rows = (q % 2) * IDX_CHUNK + g * LANES + lanes
                off = q * IDX_CHUNK + g * LANES
                x = xv[pl.ds(off, LANES)]
                denom = jnp.zeros((LANES,), jnp.float32)
                numer = jnp.zeros((LANES,), jnp.float32)
                for d in range(LATENT):
                    dd = jnp.full((LANES,), d, jnp.int32)
                    e = plsc.load_gather(ibuf, [rows, dd])
                    s = plsc.load_gather(ubuf, [rows, dd])
                    denom = denom + e
                    numer = numer + s * e
                for d in range(XDIM):
                    d1 = jnp.full((LANES,), LATENT + d, jnp.int32)
                    d0 = jnp.full((LANES,), LATENT + XDIM + d, jnp.int32)
                    x1 = plsc.load_gather(ibuf, [rows, d1])
                    x0 = plsc.load_gather(ibuf, [rows, d0])
                    e = jnp.exp(x1 * x + x0 * (one - x))
                    s = plsc.load_gather(ubuf, [rows, d1])
                    denom = denom + e
                    numer = numer + s * e
                outv[pl.ds(off, LANES)] = numer / denom
                return carry

            lax.fori_loop(0, IDX_CHUNK // LANES, group_body, 0)

        sems = (sem_a, sem_b)
        pend = fire(0, sems[0])
        for q in range(NCHUNK):
            for cp in pend:
                cp.wait()
            if q + 1 < NCHUNK:
                pend = fire(q + 1, sems[(q + 1) % 2])
            compute(q)

        pltpu.sync_copy(outv, out_h.at[pl.ds(base, RPW)])

    return sc_kernel(users.astype(jnp.int32), items, xij, us_p, icat)


# speedup vs baseline: 5.7100x; 1.0099x over previous
import functools

import jax
import jax.numpy as jnp
from jax import lax
from jax.experimental import pallas as pl
from jax.experimental.pallas import tpu as pltpu
from jax.experimental.pallas import tpu_sc as plsc

LATENT = 64
XDIM = 16
UDIM = LATENT + XDIM
LANES = 16
WIDTH = 128
IDX_CHUNK = 128
HALF = 256


def kernel(users, items, xij, emb_user, emb_item, emb_item_xij1, emb_item_xij0):
    B = users.shape[0]
    NI = emb_item.shape[0]
    info = plsc.get_sparse_core_info()
    NC, NS = info.num_cores, info.num_subcores
    NW = NC * NS
    assert B % (NW * LANES) == 0
    RPW = B // NW
    NCHUNK = RPW // IDX_CHUNK
    NHALF = RPW // HALF
    CPH = HALF // IDX_CHUNK

    proj_u = jnp.eye(UDIM, WIDTH, dtype=jnp.float32)
    us_p = jax.nn.sigmoid(emb_user) @ proj_u
    proj_i = jnp.eye(3 * LATENT // 2, WIDTH, dtype=jnp.float32)
    icat = jnp.concatenate(
        [jnp.exp(emb_item), emb_item_xij1, emb_item_xij0], axis=1) @ proj_i

    mesh = plsc.VectorSubcoreMesh(core_axis_name="c", subcore_axis_name="s")

    @functools.partial(
        pl.kernel,
        out_type=jax.ShapeDtypeStruct((B,), jnp.float32),
        mesh=mesh,
        scratch_types=[
            pltpu.VMEM((RPW,), jnp.int32),
            pltpu.VMEM((RPW,), jnp.int32),
            pltpu.VMEM((RPW,), jnp.float32),
            pltpu.VMEM((2 * IDX_CHUNK, WIDTH), jnp.float32),
            pltpu.VMEM((2 * IDX_CHUNK, WIDTH), jnp.float32),
            pltpu.VMEM((RPW,), jnp.float32),
            pltpu.SemaphoreType.DMA,
            pltpu.SemaphoreType.DMA,
        ],
        compiler_params=pltpu.CompilerParams(
            needs_layout_passes=False, use_tc_tiling_on_sc=True),
    )
    def sc_kernel(users_h, items_h, xij_h, eu_h, ic_h, out_h,
                  uidx, iidx, xv, ubuf, ibuf, outv, sem_a, sem_b):
        wid = lax.axis_index("s") * NC + lax.axis_index("c")
        base = wid * RPW

        bsl = pl.ds(base, RPW)
        c1 = pltpu.async_copy(users_h.at[bsl], uidx, sem_a)
        c2 = pltpu.async_copy(items_h.at[bsl], iidx, sem_a)
        c3 = pltpu.async_copy(xij_h.at[bsl], xv, sem_a)
        c1.wait()
        c2.wait()
        c3.wait()

        lanes = lax.broadcasted_iota(jnp.int32, (LANES,), 0)
        one = jnp.float32(1.0)

        def fire(q, sem):
            sl = pl.ds((q % 2) * IDX_CHUNK, IDX_CHUNK)
            qsl = pl.ds(q * IDX_CHUNK, IDX_CHUNK)
            cu = pltpu.async_copy(eu_h.at[uidx.at[qsl]], ubuf.at[sl], sem)
            ci = pltpu.async_copy(ic_h.at[iidx.at[qsl]], ibuf.at[sl], sem)
            return (cu, ci)

        def compute(q):
            def group_body(g, carry):
                rows = (q % 2) * IDX_CHUNK + g * LANES + lanes
                off = q * IDX_CHUNK + g * LANES
                x = xv[pl.ds(off, LANES)]
                denom = jnp.zeros((LANES,), jnp.float32)
                numer = jnp.zeros((LANES,), jnp.float32)
                for d in range(LATENT):
                    dd = jnp.full((LANES,), d, jnp.int32)
                    e = plsc.load_gather(ibuf, [rows, dd])
                    s = plsc.load_gather(ubuf, [rows, dd])
                    denom = denom + e
                    numer = numer + s * e
                for d in range(XDIM):
                    d1 = jnp.full((LANES,), LATENT + d, jnp.int32)
                    d0 = jnp.full((LANES,), LATENT + XDIM + d, jnp.int32)
                    x1 = plsc.load_gather(ibuf, [rows, d1])
                    x0 = plsc.load_gather(ibuf, [rows, d0])
                    e = jnp.exp(x1 * x + x0 * (one - x))
                    s = plsc.load_gather(ubuf, [rows, d1])
                    denom = denom + e
                    numer = numer + s * e
                outv[pl.ds(off, LANES)] = numer / denom
                return carry

            lax.fori_loop(0, IDX_CHUNK // LANES, group_body, 0)

        sems = (sem_a, sem_b)
        pend = fire(0, sems[0])
        for q in range(NCHUNK):
            for cp in pend:
                cp.wait()
            if q + 1 < NCHUNK:
                pend = fire(q + 1, sems[(q + 1) % 2])
            compute(q)

        pltpu.sync_copy(outv, out_h.at[pl.ds(base, RPW)])

    return sc_kernel(users.astype(jnp.int32), items, xij, us_p, icat)
